# Initial kernel scaffold; baseline (speedup 1.0000x reference)
import jax
import jax.numpy as jnp
from jax.experimental import pallas as pl


def kernel(x, params):
    raise NotImplementedError("write your pallas kernel here")


# speedup vs baseline: 1.2800x; 1.2800x over previous
import jax
import jax.numpy as jnp
from jax.experimental import pallas as pl

V = 256
D = 512
L = 4
H = 8
DH = D // H
B = 32
T = 512
NT = 64
K = 8
DT = 32
N = B * T
R = 512
NBLK = N // R
LN_EPS = 1e-5


def _ln(h, s, b):
    m = h.mean(-1, keepdims=True)
    v = ((h - m) ** 2).mean(-1, keepdims=True)
    return (h - m) / jnp.sqrt(v + LN_EPS) * s + b


def _embed_kernel(x_ref, emb_ref, pos_ref, o_ref):
    ids = x_ref[0, 0]
    onehot = (ids[:, None] == jax.lax.broadcasted_iota(jnp.int32, (T, V), 1))
    o_ref[0] = onehot.astype(jnp.float32) @ emb_ref[...] + pos_ref[...]


def _qkv_kernel(h_ref, s_ref, b_ref, w_ref, bias_ref, o_ref):
    hn = _ln(h_ref[...], s_ref[...], b_ref[...])
    qkv = hn @ w_ref[...] + bias_ref[...]
    o_ref[0] = qkv.reshape(T, 3 * H, DH).transpose(1, 0, 2)


def _attn_kernel(q_ref, k_ref, v_ref, o_ref):
    q = q_ref[0, 0]
    k = k_ref[0, 0]
    v = v_ref[0, 0]
    s = jax.lax.dot_general(q, k, (((1,), (1,)), ((), ()))) * 0.125
    ri = jax.lax.broadcasted_iota(jnp.int32, (T, T), 0)
    ci = jax.lax.broadcasted_iota(jnp.int32, (T, T), 1)
    s = jnp.where(ri >= ci, s, -1e9)
    p = jax.nn.softmax(s, axis=-1)
    o_ref[0, 0] = p @ v


def _ffn_kernel(h_ref, attn_ref, wo_ref, bo_ref, s2_ref, b2ln_ref, wr_ref,
                w1_ref, b1_ref, w2_ref, b2_ref,
                o_ref, imp_ref, load_ref):
    pid = pl.program_id(0)
    attn = attn_ref[0].transpose(1, 0, 2).reshape(T, D)
    h1 = h_ref[...] + attn @ wo_ref[...] + bo_ref[...]
    hn2 = _ln(h1, s2_ref[...], b2ln_ref[...])
    rlog = hn2 @ wr_ref[...]
    m0 = rlog.max(-1, keepdims=True)
    col = jax.lax.broadcasted_iota(jnp.int32, (R, NT), 1)
    work = rlog
    gates_u = jnp.zeros_like(rlog)
    for _ in range(K):
        cm = work.max(-1, keepdims=True)
        eq = work == cm
        fidx = jnp.where(eq, col, NT).min(-1, keepdims=True)
        first = col == fidx
        gates_u = gates_u + jnp.where(first, jnp.exp(work - m0), 0.0)
        work = jnp.where(first, -jnp.inf, work)
    gates = gates_u / gates_u.sum(-1, keepdims=True)
    hidden = jnp.maximum(hn2 @ w1_ref[...] + b1_ref[...], 0.0)
    gated = (hidden.reshape(R, NT, DT) * gates[:, :, None]).reshape(R, NT * DT)
    ffn = gated @ w2_ref[...] + b2_ref[...]
    o_ref[...] = h1 + ffn
    probs = jax.nn.softmax(rlog, axis=-1)
    imp_part = probs.sum(0, keepdims=True)
    load_part = (gates > 0).astype(jnp.float32).sum(0, keepdims=True)

    @pl.when(pid == 0)
    def _():
        imp_ref[...] = jnp.zeros_like(imp_ref)
        load_ref[...] = jnp.zeros_like(load_ref)

    imp_ref[...] += imp_part
    load_ref[...] += load_part


def _head_kernel(h_ref, s_ref, b_ref, w_ref, bias_ref, o_ref):
    hn = _ln(h_ref[...], s_ref[...], b_ref[...])
    o_ref[...] = hn @ w_ref[...] + bias_ref[...]


def _row2(v):
    return v.reshape(1, -1)


@jax.jit
def _forward(x, params):
    x3 = x.reshape(B, 1, T).astype(jnp.int32)
    h = pl.pallas_call(
        _embed_kernel,
        grid=(B,),
        in_specs=[
            pl.BlockSpec((1, 1, T), lambda b: (b, 0, 0)),
            pl.BlockSpec((V, D), lambda b: (0, 0)),
            pl.BlockSpec((T, D), lambda b: (0, 0)),
        ],
        out_specs=pl.BlockSpec((1, T, D), lambda b: (b, 0, 0)),
        out_shape=jax.ShapeDtypeStruct((B, T, D), jnp.float32),
    )(x3, params['embedding'], params['pos_embedding'][:T]).reshape(N, D)

    total_aux = jnp.float32(0.0)
    for lp in params['layers']:
        qkv = pl.pallas_call(
            _qkv_kernel,
            grid=(NBLK,),
            in_specs=[
                pl.BlockSpec((R, D), lambda i: (i, 0)),
                pl.BlockSpec((1, D), lambda i: (0, 0)),
                pl.BlockSpec((1, D), lambda i: (0, 0)),
                pl.BlockSpec((D, 3 * D), lambda i: (0, 0)),
                pl.BlockSpec((1, 3 * D), lambda i: (0, 0)),
            ],
            out_specs=pl.BlockSpec((1, 3 * H, T, DH), lambda i: (i, 0, 0, 0)),
            out_shape=jax.ShapeDtypeStruct((B, 3 * H, T, DH), jnp.float32),
        )(h, _row2(lp['ln1_s']), _row2(lp['ln1_b']), lp['wqkv'],
          _row2(lp['bqkv']))

        attn = pl.pallas_call(
            _attn_kernel,
            grid=(B, H),
            in_specs=[
                pl.BlockSpec((1, 1, T, DH), lambda b, hh: (b, hh, 0, 0)),
                pl.BlockSpec((1, 1, T, DH), lambda b, hh: (b, H + hh, 0, 0)),
                pl.BlockSpec((1, 1, T, DH), lambda b, hh: (b, 2 * H + hh, 0, 0)),
            ],
            out_specs=pl.BlockSpec((1, 1, T, DH), lambda b, hh: (b, hh, 0, 0)),
            out_shape=jax.ShapeDtypeStruct((B, H, T, DH), jnp.float32),
        )(qkv, qkv, qkv)

        h, imp, load = pl.pallas_call(
            _ffn_kernel,
            grid=(NBLK,),
            in_specs=[
                pl.BlockSpec((R, D), lambda i: (i, 0)),
                pl.BlockSpec((1, H, T, DH), lambda i: (i, 0, 0, 0)),
                pl.BlockSpec((D, D), lambda i: (0, 0)),
                pl.BlockSpec((1, D), lambda i: (0, 0)),
                pl.BlockSpec((1, D), lambda i: (0, 0)),
                pl.BlockSpec((1, D), lambda i: (0, 0)),
                pl.BlockSpec((D, NT), lambda i: (0, 0)),
                pl.BlockSpec((D, NT * DT), lambda i: (0, 0)),
                pl.BlockSpec((1, NT * DT), lambda i: (0, 0)),
                pl.BlockSpec((NT * DT, D), lambda i: (0, 0)),
                pl.BlockSpec((1, D), lambda i: (0, 0)),
            ],
            out_specs=[
                pl.BlockSpec((R, D), lambda i: (i, 0)),
                pl.BlockSpec((1, NT), lambda i: (0, 0)),
                pl.BlockSpec((1, NT), lambda i: (0, 0)),
            ],
            out_shape=[
                jax.ShapeDtypeStruct((N, D), jnp.float32),
                jax.ShapeDtypeStruct((1, NT), jnp.float32),
                jax.ShapeDtypeStruct((1, NT), jnp.float32),
            ],
        )(h, attn, lp['wo'], _row2(lp['bo']), _row2(lp['ln2_s']),
          _row2(lp['ln2_b']), lp['wr'], lp['w1'], _row2(lp['b1']),
          lp['w2'].reshape(NT * DT, D), _row2(lp['b2']))
        total_aux = total_aux + NT * jnp.sum(
            (imp[0] / N) * (load[0] / N))

    logits = pl.pallas_call(
        _head_kernel,
        grid=(NBLK,),
        in_specs=[
            pl.BlockSpec((R, D), lambda i: (i, 0)),
            pl.BlockSpec((1, D), lambda i: (0, 0)),
            pl.BlockSpec((1, D), lambda i: (0, 0)),
            pl.BlockSpec((D, V), lambda i: (0, 0)),
            pl.BlockSpec((1, V), lambda i: (0, 0)),
        ],
        out_specs=pl.BlockSpec((R, V), lambda i: (i, 0)),
        out_shape=jax.ShapeDtypeStruct((N, V), jnp.float32),
    )(h, _row2(params['lnf_s']), _row2(params['lnf_b']),
      params['head_w'], _row2(params['head_b'])).reshape(B, T, V)

    return logits, total_aux


def kernel(x, params):
    return _forward(x, params)


# speedup vs baseline: 3.5827x; 2.7991x over previous
import jax
import jax.numpy as jnp
from jax.experimental import pallas as pl

V = 256
D = 512
L = 4
H = 8
DH = D // H
B = 32
T = 512
NT = 64
K = 8
DT = 32
N = B * T
R = 512
NBLK = N // R
LN_EPS = 1e-5


def _ln(h, s, b):
    m = h.mean(-1, keepdims=True)
    v = ((h - m) ** 2).mean(-1, keepdims=True)
    return (h - m) / jnp.sqrt(v + LN_EPS) * s + b


def _embed_kernel(x_ref, emb_ref, pos_ref, o_ref):
    ids = x_ref[0, 0]
    onehot = (ids[:, None] == jax.lax.broadcasted_iota(jnp.int32, (T, V), 1))
    o_ref[0] = onehot.astype(jnp.float32) @ emb_ref[...] + pos_ref[...]


def _attn_kernel(h_ref, s_ref, b_ref, w_ref, bias_ref, wo_ref, bo_ref, o_ref):
    h = h_ref[...]
    hn = _ln(h, s_ref[...], b_ref[...])
    qkv = hn @ w_ref[...] + bias_ref[...]
    ri = jax.lax.broadcasted_iota(jnp.int32, (T, T), 0)
    ci = jax.lax.broadcasted_iota(jnp.int32, (T, T), 1)
    neg = jnp.float32(-1e9)
    cols = []
    for hh in range(H):
        q = qkv[:, hh * DH:(hh + 1) * DH]
        k = qkv[:, D + hh * DH:D + (hh + 1) * DH]
        v = qkv[:, 2 * D + hh * DH:2 * D + (hh + 1) * DH]
        s = jax.lax.dot_general(q, k, (((1,), (1,)), ((), ()))) * 0.125
        s = jnp.where(ri >= ci, s, neg)
        p = jax.nn.softmax(s, axis=-1)
        cols.append(p @ v)
    attn = jnp.concatenate(cols, axis=-1)
    o_ref[...] = h + attn @ wo_ref[...] + bo_ref[...]


def _ffn_kernel(h_ref, s2_ref, b2ln_ref, wr_ref, exp_ref,
                w1_ref, b1_ref, w2_ref, b2_ref,
                o_ref, imp_ref, load_ref):
    pid = pl.program_id(0)
    h1 = h_ref[...]
    hn2 = _ln(h1, s2_ref[...], b2ln_ref[...])
    rlog = hn2 @ wr_ref[...]
    m0 = rlog.max(-1, keepdims=True)
    col = jax.lax.broadcasted_iota(jnp.int32, (R, NT), 1)
    work = rlog
    gates_u = jnp.zeros_like(rlog)
    for _ in range(K):
        cm = work.max(-1, keepdims=True)
        eq = work == cm
        fidx = jnp.where(eq, col, NT).min(-1, keepdims=True)
        first = col == fidx
        gates_u = gates_u + jnp.where(first, jnp.exp(work - m0), 0.0)
        work = jnp.where(first, -jnp.inf, work)
    gates = gates_u / gates_u.sum(-1, keepdims=True)
    hidden = jnp.maximum(hn2 @ w1_ref[...] + b1_ref[...], 0.0)
    gate_exp = gates @ exp_ref[...]
    ffn = (hidden * gate_exp) @ w2_ref[...] + b2_ref[...]
    o_ref[...] = h1 + ffn
    probs = jax.nn.softmax(rlog, axis=-1)
    imp_part = probs.sum(0, keepdims=True)
    load_part = (gates > 0).astype(jnp.float32).sum(0, keepdims=True)

    @pl.when(pid == 0)
    def _():
        imp_ref[...] = jnp.zeros_like(imp_ref)
        load_ref[...] = jnp.zeros_like(load_ref)

    imp_ref[...] += imp_part
    load_ref[...] += load_part


def _head_kernel(h_ref, s_ref, b_ref, w_ref, bias_ref, o_ref):
    hn = _ln(h_ref[...], s_ref[...], b_ref[...])
    o_ref[...] = hn @ w_ref[...] + bias_ref[...]


def _row2(v):
    return v.reshape(1, -1)


@jax.jit
def _forward(x, params):
    x3 = x.reshape(B, 1, T).astype(jnp.int32)
    h = pl.pallas_call(
        _embed_kernel,
        grid=(B,),
        in_specs=[
            pl.BlockSpec((1, 1, T), lambda b: (b, 0, 0)),
            pl.BlockSpec((V, D), lambda b: (0, 0)),
            pl.BlockSpec((T, D), lambda b: (0, 0)),
        ],
        out_specs=pl.BlockSpec((1, T, D), lambda b: (b, 0, 0)),
        out_shape=jax.ShapeDtypeStruct((B, T, D), jnp.float32),
    )(x3, params['embedding'], params['pos_embedding'][:T]).reshape(N, D)

    tile_of_col = jnp.arange(NT * DT, dtype=jnp.int32) // DT
    expand = (tile_of_col[None, :] ==
              jnp.arange(NT, dtype=jnp.int32)[:, None]).astype(jnp.float32)

    total_aux = jnp.float32(0.0)
    for lp in params['layers']:
        h = pl.pallas_call(
            _attn_kernel,
            grid=(B,),
            in_specs=[
                pl.BlockSpec((T, D), lambda i: (i, 0)),
                pl.BlockSpec((1, D), lambda i: (0, 0)),
                pl.BlockSpec((1, D), lambda i: (0, 0)),
                pl.BlockSpec((D, 3 * D), lambda i: (0, 0)),
                pl.BlockSpec((1, 3 * D), lambda i: (0, 0)),
                pl.BlockSpec((D, D), lambda i: (0, 0)),
                pl.BlockSpec((1, D), lambda i: (0, 0)),
            ],
            out_specs=pl.BlockSpec((T, D), lambda i: (i, 0)),
            out_shape=jax.ShapeDtypeStruct((N, D), jnp.float32),
        )(h, _row2(lp['ln1_s']), _row2(lp['ln1_b']), lp['wqkv'],
          _row2(lp['bqkv']), lp['wo'], _row2(lp['bo']))

        h, imp, load = pl.pallas_call(
            _ffn_kernel,
            grid=(NBLK,),
            in_specs=[
                pl.BlockSpec((R, D), lambda i: (i, 0)),
                pl.BlockSpec((1, D), lambda i: (0, 0)),
                pl.BlockSpec((1, D), lambda i: (0, 0)),
                pl.BlockSpec((D, NT), lambda i: (0, 0)),
                pl.BlockSpec((NT, NT * DT), lambda i: (0, 0)),
                pl.BlockSpec((D, NT * DT), lambda i: (0, 0)),
                pl.BlockSpec((1, NT * DT), lambda i: (0, 0)),
                pl.BlockSpec((NT * DT, D), lambda i: (0, 0)),
                pl.BlockSpec((1, D), lambda i: (0, 0)),
            ],
            out_specs=[
                pl.BlockSpec((R, D), lambda i: (i, 0)),
                pl.BlockSpec((1, NT), lambda i: (0, 0)),
                pl.BlockSpec((1, NT), lambda i: (0, 0)),
            ],
            out_shape=[
                jax.ShapeDtypeStruct((N, D), jnp.float32),
                jax.ShapeDtypeStruct((1, NT), jnp.float32),
                jax.ShapeDtypeStruct((1, NT), jnp.float32),
            ],
        )(h, _row2(lp['ln2_s']), _row2(lp['ln2_b']), lp['wr'], expand,
          lp['w1'], _row2(lp['b1']), lp['w2'].reshape(NT * DT, D),
          _row2(lp['b2']))
        total_aux = total_aux + NT * jnp.sum(
            (imp[0] / N) * (load[0] / N))

    logits = pl.pallas_call(
        _head_kernel,
        grid=(NBLK,),
        in_specs=[
            pl.BlockSpec((R, D), lambda i: (i, 0)),
            pl.BlockSpec((1, D), lambda i: (0, 0)),
            pl.BlockSpec((1, D), lambda i: (0, 0)),
            pl.BlockSpec((D, V), lambda i: (0, 0)),
            pl.BlockSpec((1, V), lambda i: (0, 0)),
        ],
        out_specs=pl.BlockSpec((R, V), lambda i: (i, 0)),
        out_shape=jax.ShapeDtypeStruct((N, V), jnp.float32),
    )(h, _row2(params['lnf_s']), _row2(params['lnf_b']),
      params['head_w'], _row2(params['head_b'])).reshape(B, T, V)

    return logits, total_aux


def kernel(x, params):
    return _forward(x, params)
